# baseline (speedup 1.0000x reference)
import functools

import jax
import jax.numpy as jnp
from jax import lax
from jax.experimental import pallas as pl
from jax.experimental.pallas import tpu as pltpu

_LOG2PI = 1.8378770664093453


def _gmm_body(z_ref, mu_ref, lv_ref, logw_ref, out_ref, prec_ref, cst_ref):
    z = z_ref[...]
    lv = lv_ref[...]
    prec_ref[...] = 0.5 * jnp.exp(-lv)
    cst_ref[...] = (-0.5 * _LOG2PI) - 0.5 * lv + logw_ref[...]

    def lp(k):
        d = z - mu_ref[k, :]
        return cst_ref[k, :] - prec_ref[k, :] * (d * d)

    def pass1(k, m):
        return jnp.maximum(m, lp(k))

    m = lax.fori_loop(1, 64, pass1, lp(0))

    def pass2(k, s):
        return s + jnp.exp(lp(k) - m)

    s = lax.fori_loop(1, 64, pass2, jnp.exp(lp(0) - m))
    out_ref[...] = m + jnp.log(s)


@jax.jit
def kernel(z, means, logvars, w):
    B, L = z.shape
    K = means.shape[0]
    wf = w.reshape(K)
    logw = wf - jax.scipy.special.logsumexp(wf)

    z2 = z.reshape(B // 2, 2 * L)
    mu2 = jnp.concatenate([means, means], axis=1)
    lv2 = jnp.concatenate([logvars, logvars], axis=1)
    logw2 = jnp.broadcast_to(logw[:, None], (K, 2 * L))

    BB = 512
    grid = (B // 2) // BB

    out2 = pl.pallas_call(
        _gmm_body,
        grid=(grid,),
        in_specs=[
            pl.BlockSpec((BB, 2 * L), lambda i: (i, 0)),
            pl.BlockSpec((K, 2 * L), lambda i: (0, 0)),
            pl.BlockSpec((K, 2 * L), lambda i: (0, 0)),
            pl.BlockSpec((K, 2 * L), lambda i: (0, 0)),
        ],
        out_specs=pl.BlockSpec((BB, 2 * L), lambda i: (i, 0)),
        out_shape=jax.ShapeDtypeStruct((B // 2, 2 * L), jnp.float32),
        scratch_shapes=[
            pltpu.VMEM((K, 2 * L), jnp.float32),
            pltpu.VMEM((K, 2 * L), jnp.float32),
        ],
    )(z2, mu2, lv2, logw2)
    return out2.reshape(B, L)


# speedup vs baseline: 1.5962x; 1.5962x over previous
import jax
import jax.numpy as jnp
from jax import lax
from jax.experimental import pallas as pl
from jax.experimental.pallas import tpu as pltpu

_LOG2PI = 1.8378770664093453
_LOG2E = 1.4426950408889634
_LN2 = 0.6931471805599453


def _gmm_body(z_ref, mu_ref, lv_ref, logw_ref, out_ref, a_ref, b_ref, c_ref):
    @pl.when(pl.program_id(0) == 0)
    def _prep():
        lv = lv_ref[...]
        mu = mu_ref[...]
        prec = 0.5 * jnp.exp(-lv)
        cst = (-0.5 * _LOG2PI) - 0.5 * lv + logw_ref[...]
        a = -prec * _LOG2E
        b = (2.0 * _LOG2E) * prec * mu
        c = (cst - prec * mu * mu) * _LOG2E
        K = lv.shape[0]
        a_ref[...] = jnp.broadcast_to(a[:, None, :], (K, 8, 128))
        b_ref[...] = jnp.broadcast_to(b[:, None, :], (K, 8, 128))
        c_ref[...] = jnp.broadcast_to(c[:, None, :], (K, 8, 128))

    z = z_ref[...]
    z2 = z * z

    def lp(k):
        return a_ref[k][None] * z2 + (b_ref[k][None] * z + c_ref[k][None])

    def pass1(k, m):
        return jnp.maximum(m, lp(k))

    m = lax.fori_loop(1, 64, pass1, lp(0), unroll=True)

    def pass2(k, s):
        return s + jnp.exp2(lp(k) - m)

    s = lax.fori_loop(1, 64, pass2, jnp.exp2(lp(0) - m), unroll=True)
    out_ref[...] = _LN2 * m + jnp.log(s)


@jax.jit
def kernel(z, means, logvars, w):
    B, L = z.shape
    K = means.shape[0]
    wf = w.reshape(K)
    logw = wf - jax.scipy.special.logsumexp(wf)

    R = 8
    z3 = z.reshape(B // 2 // R, R, 2 * L)
    mu2 = jnp.concatenate([means, means], axis=1)
    lv2 = jnp.concatenate([logvars, logvars], axis=1)
    logw2 = jnp.broadcast_to(logw[:, None], (K, 2 * L))

    grid = B // 2 // R // R

    out3 = pl.pallas_call(
        _gmm_body,
        grid=(grid,),
        in_specs=[
            pl.BlockSpec((R, R, 2 * L), lambda i: (i, 0, 0)),
            pl.BlockSpec((K, 2 * L), lambda i: (0, 0)),
            pl.BlockSpec((K, 2 * L), lambda i: (0, 0)),
            pl.BlockSpec((K, 2 * L), lambda i: (0, 0)),
        ],
        out_specs=pl.BlockSpec((R, R, 2 * L), lambda i: (i, 0, 0)),
        out_shape=jax.ShapeDtypeStruct((B // 2 // R, R, 2 * L), jnp.float32),
        scratch_shapes=[
            pltpu.VMEM((K, R, 2 * L), jnp.float32),
            pltpu.VMEM((K, R, 2 * L), jnp.float32),
            pltpu.VMEM((K, R, 2 * L), jnp.float32),
        ],
    )(z3, mu2, lv2, logw2)
    return out3.reshape(B, L)
